# Initial kernel scaffold; baseline (speedup 1.0000x reference)
import functools

import jax
import jax.numpy as jnp
from jax import lax
from jax.experimental import pallas as pl
from jax.experimental.pallas import tpu as pltpu
from jax.experimental.pallas import tpu_sc as plsc

VOCAB = 1000000
D_EMB = 64
B = 4096
L = 200

N = B * L
NW = 32
N_PER_W = N // NW
CHUNK = 512
N_CHUNKS = N_PER_W // CHUNK
DMA_IDX = 128
DMA_PER_CHUNK = CHUNK // DMA_IDX


def _embed_body(x_hbm, mask_hbm, table_hbm, out_hbm,
                x_v, m_v, idx_v, rows_v, sem):
    wid = lax.axis_index("s") * 2 + lax.axis_index("c")
    lane = lax.iota(jnp.int32, 16)
    zeros16 = jnp.zeros((16,), jnp.float32)

    @pl.loop(0, N_CHUNKS)
    def _chunk(c):
        base = wid * N_PER_W + c * CHUNK
        pltpu.sync_copy(x_hbm.at[pl.ds(base, CHUNK)], x_v)
        pltpu.sync_copy(mask_hbm.at[pl.ds(base, CHUNK)], m_v)

        for i in range(DMA_PER_CHUNK):
            for g in range(DMA_IDX // 16):
                s = i * DMA_IDX + g * 16
                idx_v[i, pl.ds(g * 16, 16)] = (
                    x_v[pl.ds(s, 16)] * m_v[pl.ds(s, 16)])

        copies = []
        for i in range(DMA_PER_CHUNK):
            copies.append(pltpu.async_copy(
                table_hbm.at[idx_v.at[i]],
                rows_v.at[pl.ds(i * DMA_IDX, DMA_IDX)],
                sem))
        for cp in copies:
            cp.wait()

        @pl.loop(0, CHUNK // 16)
        def _zero(g):
            m16 = m_v[pl.ds(g * 16, 16)]
            pred = m16 == 0
            row_ids = g * 16 + lane
            for j in range(D_EMB):
                plsc.store_scatter(
                    rows_v, [row_ids, jnp.full((16,), j, jnp.int32)],
                    zeros16, mask=pred)

        pltpu.sync_copy(rows_v, out_hbm.at[pl.ds(base, CHUNK)])


@jax.jit
def _embed(x_flat, mask_flat, embed_weight):
    mesh = plsc.VectorSubcoreMesh(core_axis_name="c", subcore_axis_name="s")
    f = pl.kernel(
        _embed_body,
        out_type=jax.ShapeDtypeStruct((N, D_EMB), jnp.float32),
        mesh=mesh,
        scratch_types=[
            pltpu.VMEM((CHUNK,), jnp.int32),
            pltpu.VMEM((CHUNK,), jnp.int32),
            pltpu.VMEM((DMA_PER_CHUNK, DMA_IDX), jnp.int32),
            pltpu.VMEM((CHUNK, D_EMB), jnp.float32),
            pltpu.SemaphoreType.DMA,
        ],
    )
    return f(x_flat, mask_flat, embed_weight)


def kernel(x, mask, embed_weight):
    x_flat = x.reshape(-1).astype(jnp.int32)
    mask_flat = mask.reshape(-1).astype(jnp.int32)
    out = _embed(x_flat, mask_flat, embed_weight)
    return out.reshape(B, L, D_EMB)


# speedup vs baseline: 1.8194x; 1.8194x over previous
import functools

import jax
import jax.numpy as jnp
from jax import lax
from jax.experimental import pallas as pl
from jax.experimental.pallas import tpu as pltpu
from jax.experimental.pallas import tpu_sc as plsc

VOCAB = 1000000
D_EMB = 64
B = 4096
L = 200

N = B * L
NW = 32
N_PER_W = N // NW
CHUNK = 512
N_CHUNKS = N_PER_W // CHUNK
DMA_IDX = 128
DMA_PER_CHUNK = CHUNK // DMA_IDX


def _embed_body(x_hbm, mask_hbm, table_hbm, out_hbm,
                x_v, m_v, idx_v, rows_v, sem):
    wid = lax.axis_index("s") * 2 + lax.axis_index("c")
    lane = lax.iota(jnp.int32, 16)
    zeros16 = jnp.zeros((16,), jnp.float32)

    @pl.loop(0, N_CHUNKS)
    def _chunk(c):
        base = wid * N_PER_W + c * CHUNK
        pltpu.sync_copy(x_hbm.at[pl.ds(base, CHUNK)], x_v)
        pltpu.sync_copy(mask_hbm.at[pl.ds(base, CHUNK)], m_v)

        for i in range(DMA_PER_CHUNK):
            for g in range(DMA_IDX // 16):
                s = i * DMA_IDX + g * 16
                idx_v[i, pl.ds(g * 16, 16)] = (
                    x_v[pl.ds(s, 16)] * m_v[pl.ds(s, 16)])

        copies = []
        for i in range(DMA_PER_CHUNK):
            copies.append(pltpu.async_copy(
                table_hbm.at[idx_v.at[i]],
                rows_v.at[pl.ds(i * DMA_IDX, DMA_IDX)],
                sem))
        for cp in copies:
            cp.wait()

        @pl.loop(0, CHUNK // 16)
        def _zero(g):
            m16 = m_v[pl.ds(g * 16, 16)]
            pred = m16 == 0
            row_ids = g * 16 + lane
            for j in range(D_EMB):
                plsc.store_scatter(
                    rows_v, [row_ids, jnp.full((16,), j, jnp.int32)],
                    zeros16, mask=pred)

        pltpu.sync_copy(rows_v, out_hbm.at[pl.ds(base, CHUNK)])


@jax.jit
def _embed(x_flat, mask_flat, embed_weight):
    mesh = plsc.VectorSubcoreMesh(core_axis_name="c", subcore_axis_name="s")
    f = pl.kernel(
        _embed_body,
        out_type=jax.ShapeDtypeStruct((N, D_EMB), jnp.float32),
        mesh=mesh,
        scratch_types=[
            pltpu.VMEM((CHUNK,), jnp.int32),
            pltpu.VMEM((CHUNK,), jnp.int32),
            pltpu.VMEM((DMA_PER_CHUNK, DMA_IDX), jnp.int32),
            pltpu.VMEM((CHUNK, D_EMB), jnp.float32),
            pltpu.SemaphoreType.DMA,
        ],
        compiler_params=pltpu.CompilerParams(
            needs_layout_passes=False, use_tc_tiling_on_sc=False),
    )
    return f(x_flat, mask_flat, embed_weight)


def kernel(x, mask, embed_weight):
    x_flat = x.reshape(-1).astype(jnp.int32)
    mask_flat = mask.reshape(-1).astype(jnp.int32)
    out = _embed(x_flat, mask_flat, embed_weight)
    return out.reshape(B, L, D_EMB)


# speedup vs baseline: 1.8205x; 1.0006x over previous
import jax
import jax.numpy as jnp
from jax import lax
from jax.experimental import pallas as pl
from jax.experimental.pallas import tpu as pltpu
from jax.experimental.pallas import tpu_sc as plsc

VOCAB = 1000000
D_EMB = 64
B = 4096
L = 200

N = B * L
NW = 32
N_PER_W = N // NW
CHUNK = 256
N_CHUNKS = N_PER_W // CHUNK
DMA_IDX = 128
DMA_PER_CHUNK = CHUNK // DMA_IDX
NBUF = 4


def _embed_body(x_hbm, mask_hbm, table_hbm, out_hbm,
                xv, mv, rows0, rows1, rows2, rows3,
                g0, g1, g2, g3, w0, w1, w2, w3):
    wid = lax.axis_index("s") * 2 + lax.axis_index("c")
    w_base = wid * N_PER_W
    lane = lax.iota(jnp.int32, 16)
    zeros16 = jnp.zeros((16,), jnp.float32)
    bufs = [(rows0, g0, w0), (rows1, g1, w1),
            (rows2, g2, w2), (rows3, g3, w3)]

    pltpu.sync_copy(x_hbm.at[pl.ds(w_base, N_PER_W)], xv)
    pltpu.sync_copy(mask_hbm.at[pl.ds(w_base, N_PER_W)], mv)

    @pl.loop(0, N_PER_W // 16, unroll=8)
    def _mul(i):
        s = i * 16
        xv[pl.ds(s, 16)] = xv[pl.ds(s, 16)] * mv[pl.ds(s, 16)]

    def gather(c, rows, sem):
        for i in range(DMA_PER_CHUNK):
            pltpu.async_copy(
                table_hbm.at[xv.at[pl.ds(c * CHUNK + i * DMA_IDX, DMA_IDX)]],
                rows.at[pl.ds(i * DMA_IDX, DMA_IDX)],
                sem)

    def gather_wait(c, rows, sem):
        for i in range(DMA_PER_CHUNK):
            pltpu.make_async_copy(
                table_hbm.at[xv.at[pl.ds(c * CHUNK + i * DMA_IDX, DMA_IDX)]],
                rows.at[pl.ds(i * DMA_IDX, DMA_IDX)],
                sem).wait()

    def out_ref(c):
        return out_hbm.at[pl.ds(w_base + c * CHUNK, CHUNK)]

    for b in range(NBUF):
        gather(b, bufs[b][0], bufs[b][1])

    @pl.loop(0, N_CHUNKS, step=NBUF)
    def _ring(c0):
        for b in range(NBUF):
            rows, gsem, wsem = bufs[b]
            c = c0 + b
            gather_wait(c, rows, gsem)

            @pl.loop(0, CHUNK // 16)
            def _zero(g):
                m16 = mv[pl.ds(c * CHUNK + g * 16, 16)]
                pred = m16 == 0
                row_ids = g * 16 + lane
                for j in range(D_EMB):
                    plsc.store_scatter(
                        rows, [row_ids, jnp.full((16,), j, jnp.int32)],
                        zeros16, mask=pred)

            pltpu.async_copy(rows, out_ref(c), wsem)

            nc = c + NBUF

            @pl.when(nc < N_CHUNKS)
            def _next():
                pltpu.make_async_copy(rows, out_ref(c), wsem).wait()
                gather(nc, rows, gsem)

            @pl.when(nc >= N_CHUNKS)
            def _last():
                pltpu.make_async_copy(rows, out_ref(c), wsem).wait()


@jax.jit
def _embed(x_flat, mask_flat, embed_weight):
    mesh = plsc.VectorSubcoreMesh(core_axis_name="c", subcore_axis_name="s")
    f = pl.kernel(
        _embed_body,
        out_type=jax.ShapeDtypeStruct((N, D_EMB), jnp.float32),
        mesh=mesh,
        scratch_types=[
            pltpu.VMEM((N_PER_W,), jnp.int32),
            pltpu.VMEM((N_PER_W,), jnp.int32),
        ] + [pltpu.VMEM((CHUNK, D_EMB), jnp.float32)] * NBUF
          + [pltpu.SemaphoreType.DMA] * (2 * NBUF),
        compiler_params=pltpu.CompilerParams(
            needs_layout_passes=False, use_tc_tiling_on_sc=False),
    )
    return f(x_flat, mask_flat, embed_weight)


def kernel(x, mask, embed_weight):
    x_flat = x.reshape(-1).astype(jnp.int32)
    mask_flat = mask.reshape(-1).astype(jnp.int32)
    out = _embed(x_flat, mask_flat, embed_weight)
    return out.reshape(B, L, D_EMB)


# speedup vs baseline: 12.4464x; 6.8367x over previous
import jax
import jax.numpy as jnp
from jax import lax
from jax.experimental import pallas as pl
from jax.experimental.pallas import tpu as pltpu
from jax.experimental.pallas import tpu_sc as plsc

VOCAB = 1000000
D_EMB = 64
B = 4096
L = 200

N = B * L
NW = 32
N_PER_W = N // NW
CHUNK = 256
N_CHUNKS = N_PER_W // CHUNK
DMA_IDX = 128
DMA_PER_CHUNK = CHUNK // DMA_IDX
NBUF = 4


def _embed_body(x_hbm, mask_hbm, table_hbm, out_hbm,
                xv, mv, rows0, rows1, rows2, rows3,
                g0, g1, g2, g3, w0, w1, w2, w3):
    wid = lax.axis_index("s") * 2 + lax.axis_index("c")
    w_base = wid * N_PER_W
    lane = lax.iota(jnp.int32, 16)
    zeros16 = jnp.zeros((16,), jnp.float32)
    bufs = [(rows0, g0, w0), (rows1, g1, w1),
            (rows2, g2, w2), (rows3, g3, w3)]

    pltpu.sync_copy(x_hbm.at[pl.ds(w_base, N_PER_W)], xv)
    pltpu.sync_copy(mask_hbm.at[pl.ds(w_base, N_PER_W)], mv)

    def gather(c, rows, sem):
        for i in range(DMA_PER_CHUNK):
            pltpu.async_copy(
                table_hbm.at[xv.at[pl.ds(c * CHUNK + i * DMA_IDX, DMA_IDX)]],
                rows.at[pl.ds(i * DMA_IDX, DMA_IDX)],
                sem)

    def gather_wait(c, rows, sem):
        for i in range(DMA_PER_CHUNK):
            pltpu.make_async_copy(
                table_hbm.at[xv.at[pl.ds(c * CHUNK + i * DMA_IDX, DMA_IDX)]],
                rows.at[pl.ds(i * DMA_IDX, DMA_IDX)],
                sem).wait()

    def out_ref(c):
        return out_hbm.at[pl.ds(w_base + c * CHUNK, CHUNK)]

    for b in range(NBUF):
        gather(b, bufs[b][0], bufs[b][1])

    @pl.loop(0, N_CHUNKS, step=NBUF)
    def _ring(c0):
        for b in range(NBUF):
            rows, gsem, wsem = bufs[b]
            c = c0 + b
            gather_wait(c, rows, gsem)

            @pl.loop(0, CHUNK // 16)
            def _zero(g):
                m16 = mv[pl.ds(c * CHUNK + g * 16, 16)]
                pred = m16 == 0
                row_ids = g * 16 + lane
                for j in range(D_EMB):
                    plsc.store_scatter(
                        rows, [row_ids, jnp.full((16,), j, jnp.int32)],
                        zeros16, mask=pred)

            pltpu.async_copy(rows, out_ref(c), wsem)

            nc = c + NBUF

            @pl.when(nc < N_CHUNKS)
            def _next():
                pltpu.make_async_copy(rows, out_ref(c), wsem).wait()
                gather(nc, rows, gsem)

            @pl.when(nc >= N_CHUNKS)
            def _last():
                pltpu.make_async_copy(rows, out_ref(c), wsem).wait()


@jax.jit
def _embed(x_flat, mask_flat, embed_weight):
    mesh = plsc.VectorSubcoreMesh(core_axis_name="c", subcore_axis_name="s")
    f = pl.kernel(
        _embed_body,
        out_type=jax.ShapeDtypeStruct((N, D_EMB), jnp.float32),
        mesh=mesh,
        scratch_types=[
            pltpu.VMEM((N_PER_W,), jnp.int32),
            pltpu.VMEM((N_PER_W,), jnp.int32),
        ] + [pltpu.VMEM((CHUNK, D_EMB), jnp.float32)] * NBUF
          + [pltpu.SemaphoreType.DMA] * (2 * NBUF),
        compiler_params=pltpu.CompilerParams(
            needs_layout_passes=False, use_tc_tiling_on_sc=False),
    )
    return f(x_flat, mask_flat, embed_weight)


def kernel(x, mask, embed_weight):
    x_flat = x.reshape(-1).astype(jnp.int32)
    mask_flat = mask.reshape(-1).astype(jnp.int32)
    out = _embed(x_flat, mask_flat, embed_weight)
    return out.reshape(B, L, D_EMB)


# speedup vs baseline: 13.0466x; 1.0482x over previous
import jax
import jax.numpy as jnp
from jax import lax
from jax.experimental import pallas as pl
from jax.experimental.pallas import tpu as pltpu
from jax.experimental.pallas import tpu_sc as plsc

VOCAB = 1000000
D_EMB = 64
B = 4096
L = 200

N = B * L
NW = 32
N_PER_W = N // NW
CHUNK = 128
N_CHUNKS = N_PER_W // CHUNK
NBUF = 8
K = 4


def _embed_body(x_hbm, mask_hbm, table_hbm, out_hbm, xv, mv, rows, sems):
    wid = lax.axis_index("s") * 2 + lax.axis_index("c")
    w_base = wid * N_PER_W
    lane = lax.iota(jnp.int32, 16)
    zeros16 = jnp.zeros((16,), jnp.float32)

    pltpu.sync_copy(x_hbm.at[pl.ds(w_base, N_PER_W)], xv)
    pltpu.sync_copy(mask_hbm.at[pl.ds(w_base, N_PER_W)], mv)

    def gather(c, b):
        pltpu.async_copy(
            table_hbm.at[xv.at[pl.ds(c * CHUNK, CHUNK)]],
            rows[b].at[pl.ds(0, CHUNK)], sems[b])

    def gather_wait(c, b):
        pltpu.make_async_copy(
            table_hbm.at[xv.at[pl.ds(c * CHUNK, CHUNK)]],
            rows[b].at[pl.ds(0, CHUNK)], sems[b]).wait()

    def wout(c, b):
        pltpu.async_copy(
            rows[b], out_hbm.at[pl.ds(w_base + c * CHUNK, CHUNK)],
            sems[NBUF + b])

    def wout_wait(c, b):
        pltpu.make_async_copy(
            rows[b], out_hbm.at[pl.ds(w_base + c * CHUNK, CHUNK)],
            sems[NBUF + b]).wait()

    for c in range(K):
        gather(c, c % NBUF)

    @pl.loop(0, N_CHUNKS, step=NBUF)
    def _ring(c0):
        for b in range(NBUF):
            c = c0 + b
            nb = (b + K) % NBUF
            nc = c + K

            @pl.when(jnp.logical_and(nc < N_CHUNKS, nc >= NBUF))
            def _refill():
                wout_wait(nc - NBUF, nb)
                gather(nc, nb)

            @pl.when(jnp.logical_and(nc < N_CHUNKS, nc < NBUF))
            def _prime():
                gather(nc, nb)

            gather_wait(c, b)

            @pl.loop(0, CHUNK // 16)
            def _zero(g):
                m16 = mv[pl.ds(c * CHUNK + g * 16, 16)]
                pred = m16 == 0
                row_ids = g * 16 + lane
                for j in range(D_EMB):
                    plsc.store_scatter(
                        rows[b], [row_ids, jnp.full((16,), j, jnp.int32)],
                        zeros16, mask=pred)

            wout(c, b)

    for t in range(NBUF):
        c = N_CHUNKS - NBUF + t
        wout_wait(c, c % NBUF)


@jax.jit
def _embed(x_flat, mask_flat, embed_weight):
    mesh = plsc.VectorSubcoreMesh(core_axis_name="c", subcore_axis_name="s")

    def body(x_hbm, mask_hbm, table_hbm, out_hbm, xv, mv, *rest):
        rows = list(rest[:NBUF])
        sems = list(rest[NBUF:])
        _embed_body(x_hbm, mask_hbm, table_hbm, out_hbm, xv, mv, rows, sems)

    f = pl.kernel(
        body,
        out_type=jax.ShapeDtypeStruct((N, D_EMB), jnp.float32),
        mesh=mesh,
        scratch_types=[
            pltpu.VMEM((N_PER_W,), jnp.int32),
            pltpu.VMEM((N_PER_W,), jnp.int32),
        ] + [pltpu.VMEM((CHUNK, D_EMB), jnp.float32)] * NBUF
          + [pltpu.SemaphoreType.DMA] * (2 * NBUF),
        compiler_params=pltpu.CompilerParams(
            needs_layout_passes=False, use_tc_tiling_on_sc=False),
    )
    return f(x_flat, mask_flat, embed_weight)


def kernel(x, mask, embed_weight):
    x_flat = x.reshape(-1).astype(jnp.int32)
    mask_flat = mask.reshape(-1).astype(jnp.int32)
    out = _embed(x_flat, mask_flat, embed_weight)
    return out.reshape(B, L, D_EMB)
